# Initial kernel scaffold; baseline (speedup 1.0000x reference)
import jax
import jax.numpy as jnp
from jax.experimental import pallas as pl


def kernel(feat1, label1, feat2, label2):
    raise NotImplementedError("write your pallas kernel here")


# speedup vs baseline: 3.2979x; 3.2979x over previous
import functools

import jax
import jax.numpy as jnp
import numpy as np
from jax import lax
from jax.experimental import pallas as pl
from jax.experimental.pallas import tpu as pltpu
from jax.experimental.pallas import tpu_sc as plsc

_B, _C, _H, _W = 4, 384, 128, 128
_HW = _H * _W
_N = _B * _HW
_SAMPLE = 512
_TEMPERATURE = 0.1
_NT = 32
_PF1 = (2 * _SAMPLE) // _NT
_PF2 = _SAMPLE // _NT


def _rotl(x, r):
    x = x.astype(np.uint32)
    return ((x << np.uint32(r)) | (x >> np.uint32(32 - r))).astype(np.uint32)


def _threefry2x32(k0, k1, x0, x1):
    x0 = np.asarray(x0, np.uint32).copy()
    x1 = np.asarray(x1, np.uint32).copy()
    ks0, ks1 = np.uint32(k0), np.uint32(k1)
    ks2 = np.uint32(ks0 ^ ks1 ^ np.uint32(0x1BD11BDA))
    rot = ((13, 15, 26, 6), (17, 29, 16, 24))
    x0 = x0 + ks0
    x1 = x1 + ks1
    sched = [(ks1, ks2), (ks2, ks0), (ks0, ks1), (ks1, ks2), (ks2, ks0)]
    for i in range(5):
        for r in rot[i % 2]:
            x0 = (x0 + x1).astype(np.uint32)
            x1 = _rotl(x1, r)
            x1 = x1 ^ x0
        a, b = sched[i]
        x0 = (x0 + a).astype(np.uint32)
        x1 = (x1 + b + np.uint32(i + 1)).astype(np.uint32)
    return x0, x1


def _np_permutation(seed, n):
    k = (np.uint32(0), np.uint32(seed))
    x = np.arange(n, dtype=np.int32)
    num_rounds = int(np.ceil(3 * np.log(max(1, n)) / np.log(2**32 - 1)))
    for _ in range(num_rounds):
        h0, h1 = _threefry2x32(k[0], k[1], np.zeros(2, np.uint32),
                               np.arange(2, dtype=np.uint32))
        k = (h0[0], h1[0])
        sub = (h0[1], h1[1])
        b1, b2 = _threefry2x32(sub[0], sub[1], np.zeros(n, np.uint32),
                               np.arange(n, dtype=np.uint32))
        order = np.argsort(b1 ^ b2, kind="stable")
        x = x[order]
    return x


@functools.lru_cache(maxsize=1)
def _plan():
    p1 = _np_permutation(1, _N)[:_SAMPLE]
    p1c = _np_permutation(2, _N)[:_SAMPLE]
    p2c = _np_permutation(3, _N)[:_SAMPLE]

    f1_pix = np.concatenate([p1, p1c]).astype(np.int64)
    f2_pix = p2c.astype(np.int64)

    c = np.arange(_C, dtype=np.int64)

    def feat_rows(pix):
        b = pix // _HW
        h = (pix % _HW) // _W
        return (b[:, None] * _C + c[None, :]) * _H + h[:, None]

    def feat_lanes(pix):
        return pix % _W

    def lab_rows_lanes(pix):
        b = pix // _HW
        h = (pix % _HW) // _W
        w = pix % _W
        return b * 512 + 4 * h, 4 * w

    lr1, ll1 = lab_rows_lanes(f1_pix)
    lr2, ll2 = lab_rows_lanes(f2_pix)

    def i32(a):
        return np.ascontiguousarray(a, dtype=np.int32)

    labmeta = np.zeros((_NT, 128), np.int64)
    labmeta[:, 0:32] = lr1.reshape(_NT, _PF1)
    labmeta[:, 32:48] = lr2.reshape(_NT, _PF2)
    labmeta[:, 48:80] = ll1.reshape(_NT, _PF1)
    labmeta[:, 80:96] = ll2.reshape(_NT, _PF2)

    return {
        "idx1": i32(feat_rows(f1_pix).reshape(_NT, _PF1 * _C)),
        "idx2": i32(feat_rows(f2_pix).reshape(_NT, _PF2 * _C)),
        "lane1": i32(np.broadcast_to(
            feat_lanes(f1_pix).reshape(_NT, _PF1, 1), (_NT, _PF1, 16)
        ).reshape(_NT, _PF1 * 16)),
        "lane2": i32(np.broadcast_to(
            feat_lanes(f2_pix).reshape(_NT, _PF2, 1), (_NT, _PF2, 16)
        ).reshape(_NT, _PF2 * 16)),
        "labmeta": i32(labmeta),
    }


def _sc_gather_body(
    f1_tab, f2_tab, l1_tab, l2_tab,
    idx1_h, idx2_h, lane1_h, lane2_h, labmeta_h,
    g_out, lab_out,
    idx1_v, idx2_v, lane1_v, lane2_v, labmeta_v,
    rows_v, labrows1_v, labrows2_v,
    out1_v, out2_v, labout_v, sem,
):
    wid = lax.axis_index("s") * 2 + lax.axis_index("c")
    iota = lax.iota(jnp.int32, 16)

    pltpu.sync_copy(idx1_h.at[wid], idx1_v)
    pltpu.sync_copy(idx2_h.at[wid], idx2_v)
    pltpu.sync_copy(lane1_h.at[wid], lane1_v)
    pltpu.sync_copy(lane2_h.at[wid], lane2_v)
    pltpu.sync_copy(labmeta_h.at[wid], labmeta_v)

    d1 = pltpu.async_copy(l1_tab.at[labmeta_v.at[pl.ds(0, _PF1)]],
                          labrows1_v, sem)
    d2 = pltpu.async_copy(l2_tab.at[labmeta_v.at[pl.ds(_PF1, _PF2)]],
                          labrows2_v, sem)
    d1.wait()
    d2.wait()
    for g in range(_PF1 // 16):
        lanes = labmeta_v[pl.ds(48 + g * 16, 16)]
        labout_v[pl.ds(g * 16, 16)] = plsc.load_gather(
            labrows1_v, [g * 16 + iota, lanes])
    for g in range(_PF2 // 16):
        lanes = labmeta_v[pl.ds(80 + g * 16, 16)]
        labout_v[pl.ds(_PF1 + g * 16, 16)] = plsc.load_gather(
            labrows2_v, [g * 16 + iota, lanes])
    pltpu.sync_copy(labout_v, lab_out.at[wid])

    def gather_pixels(tab, idx_v, lane_v, out_v, n_pix):
        def body(p, carry):
            base = pl.multiple_of(p * _C, _C)
            ds = [
                pltpu.async_copy(
                    tab.at[idx_v.at[pl.ds(base + k * 128, 128)]],
                    rows_v.at[pl.ds(k * 128, 128)],
                    sem,
                )
                for k in range(_C // 128)
            ]
            for d in ds:
                d.wait()
            lanes = lane_v[pl.ds(p * 16, 16)]

            def ext(g, carry2):
                vals = plsc.load_gather(rows_v, [g * 16 + iota, lanes])
                out_v[pl.ds(base + g * 16, 16)] = vals
                return carry2

            lax.fori_loop(0, _C // 16, ext, 0, unroll=4)
            return carry

        lax.fori_loop(0, n_pix, body, 0)

    gather_pixels(f1_tab, idx1_v, lane1_v, out1_v, _PF1)
    gather_pixels(f2_tab, idx2_v, lane2_v, out2_v, _PF2)

    pltpu.sync_copy(out1_v, g_out.at[pl.ds(wid * (_PF1 * _C), _PF1 * _C)])
    pltpu.sync_copy(
        out2_v, g_out.at[pl.ds(1024 * _C + wid * (_PF2 * _C), _PF2 * _C)])


@functools.lru_cache(maxsize=1)
def _build_sc_gather():
    mesh = plsc.VectorSubcoreMesh(core_axis_name="c", subcore_axis_name="s")
    return pl.kernel(
        _sc_gather_body,
        out_type=(
            jax.ShapeDtypeStruct((1536 * _C,), jnp.float32),
            jax.ShapeDtypeStruct((_NT, 128), jnp.int32),
        ),
        mesh=mesh,
        compiler_params=pltpu.CompilerParams(needs_layout_passes=False),
        scratch_types=[
            pltpu.VMEM((_PF1 * _C,), jnp.int32),
            pltpu.VMEM((_PF2 * _C,), jnp.int32),
            pltpu.VMEM((_PF1 * 16,), jnp.int32),
            pltpu.VMEM((_PF2 * 16,), jnp.int32),
            pltpu.VMEM((128,), jnp.int32),
            pltpu.VMEM((_C, _W), jnp.float32),
            pltpu.VMEM((_PF1, 512), jnp.int32),
            pltpu.VMEM((_PF2, 512), jnp.int32),
            pltpu.VMEM((_PF1 * _C,), jnp.float32),
            pltpu.VMEM((_PF2 * _C,), jnp.float32),
            pltpu.VMEM((128,), jnp.int32),
            pltpu.SemaphoreType.DMA,
        ],
    )


def _tc_loss_body(g_ref, alab_ref, clab_ref, out_ref):
    g = g_ref[...]
    ss = jnp.sum(g * g, axis=1, keepdims=True)
    norm = jnp.maximum(jnp.sqrt(ss), 1e-12)
    gn = g / norm
    anchors = gn[0:512, :]
    contras = gn[512:1536, :]
    logits = lax.dot_general(
        anchors, contras, (((1,), (1,)), ((), ())),
        preferred_element_type=jnp.float32,
    ) * (1.0 / _TEMPERATURE)

    a = alab_ref[...]
    c = clab_ref[...]
    eq = a == c
    ri = lax.broadcasted_iota(jnp.int32, (512, 1024), 0)
    ci = lax.broadcasted_iota(jnp.int32, (512, 1024), 1)
    mask = jnp.where(eq & (ri != ci), 1.0, 0.0)

    lmax = jnp.max(logits, axis=1, keepdims=True)
    l = logits - lmax
    e = jnp.exp(l)
    negsum = jnp.sum(e * (1.0 - mask), axis=1, keepdims=True)
    lp = l - jnp.log(e + negsum)
    msum = jnp.sum(mask, axis=1)
    msum = jnp.where(msum == 0.0, 1.0, msum)
    mlpp = jnp.sum(mask * lp, axis=1) / msum
    out_ref[0, 0] = -jnp.mean(mlpp)


def _tc_loss(g, a_lab, c_lab):
    return pl.pallas_call(
        _tc_loss_body,
        out_shape=jax.ShapeDtypeStruct((1, 1), jnp.float32),
        out_specs=pl.BlockSpec(memory_space=pltpu.SMEM),
    )(g, a_lab, c_lab)


def kernel(feat1, label1, feat2, label2):
    plan = _plan()
    g_flat, lab2d = _build_sc_gather()(
        feat1.reshape(-1, _W),
        feat2.reshape(-1, _W),
        label1.reshape(-1, 512),
        label2.reshape(-1, 512),
        plan["idx1"], plan["idx2"], plan["lane1"], plan["lane2"],
        plan["labmeta"],
    )
    g = g_flat.reshape(1536, _C)
    lab1 = lab2d[:, :_PF1].reshape(1024)
    lab2 = lab2d[:, _PF1:_PF1 + _PF2].reshape(512)
    a_lab = lab1[:512].astype(jnp.float32).reshape(512, 1)
    c_lab = jnp.concatenate([lab1[512:], lab2]).astype(
        jnp.float32).reshape(1, 1024)
    return _tc_loss(g, a_lab, c_lab)[0, 0]


# speedup vs baseline: 3.6577x; 1.1091x over previous
import functools

import jax
import jax.numpy as jnp
import numpy as np
from jax import lax
from jax.experimental import pallas as pl
from jax.experimental.pallas import tpu as pltpu
from jax.experimental.pallas import tpu_sc as plsc

_B, _C, _H, _W = 4, 384, 128, 128
_HW = _H * _W
_N = _B * _HW
_SAMPLE = 512
_TEMPERATURE = 0.1
_NT = 32
_PF1 = (2 * _SAMPLE) // _NT
_PF2 = _SAMPLE // _NT


def _rotl(x, r):
    x = x.astype(np.uint32)
    return ((x << np.uint32(r)) | (x >> np.uint32(32 - r))).astype(np.uint32)


def _threefry2x32(k0, k1, x0, x1):
    x0 = np.asarray(x0, np.uint32).copy()
    x1 = np.asarray(x1, np.uint32).copy()
    ks0, ks1 = np.uint32(k0), np.uint32(k1)
    ks2 = np.uint32(ks0 ^ ks1 ^ np.uint32(0x1BD11BDA))
    rot = ((13, 15, 26, 6), (17, 29, 16, 24))
    x0 = x0 + ks0
    x1 = x1 + ks1
    sched = [(ks1, ks2), (ks2, ks0), (ks0, ks1), (ks1, ks2), (ks2, ks0)]
    for i in range(5):
        for r in rot[i % 2]:
            x0 = (x0 + x1).astype(np.uint32)
            x1 = _rotl(x1, r)
            x1 = x1 ^ x0
        a, b = sched[i]
        x0 = (x0 + a).astype(np.uint32)
        x1 = (x1 + b + np.uint32(i + 1)).astype(np.uint32)
    return x0, x1


def _np_permutation(seed, n):
    k = (np.uint32(0), np.uint32(seed))
    x = np.arange(n, dtype=np.int32)
    num_rounds = int(np.ceil(3 * np.log(max(1, n)) / np.log(2**32 - 1)))
    for _ in range(num_rounds):
        h0, h1 = _threefry2x32(k[0], k[1], np.zeros(2, np.uint32),
                               np.arange(2, dtype=np.uint32))
        k = (h0[0], h1[0])
        sub = (h0[1], h1[1])
        b1, b2 = _threefry2x32(sub[0], sub[1], np.zeros(n, np.uint32),
                               np.arange(n, dtype=np.uint32))
        order = np.argsort(b1 ^ b2, kind="stable")
        x = x[order]
    return x


@functools.lru_cache(maxsize=1)
def _plan():
    p1 = _np_permutation(1, _N)[:_SAMPLE]
    p1c = _np_permutation(2, _N)[:_SAMPLE]
    p2c = _np_permutation(3, _N)[:_SAMPLE]

    f1_pix = np.concatenate([p1, p1c]).astype(np.int64)
    f2_pix = p2c.astype(np.int64)

    c = np.arange(_C, dtype=np.int64)

    def feat_rows(pix):
        b = pix // _HW
        h = (pix % _HW) // _W
        return (b[:, None] * _C + c[None, :]) * _H + h[:, None]

    def feat_lanes(pix):
        return pix % _W

    def lab_rows_lanes(pix):
        b = pix // _HW
        h = (pix % _HW) // _W
        w = pix % _W
        return b * 512 + 4 * h, 4 * w

    lr1, ll1 = lab_rows_lanes(f1_pix)
    lr2, ll2 = lab_rows_lanes(f2_pix)

    def i32(a):
        return np.ascontiguousarray(a, dtype=np.int32)

    labmeta = np.zeros((_NT, 128), np.int64)
    labmeta[:, 0:32] = lr1.reshape(_NT, _PF1)
    labmeta[:, 32:48] = lr2.reshape(_NT, _PF2)
    labmeta[:, 48:80] = ll1.reshape(_NT, _PF1)
    labmeta[:, 80:96] = ll2.reshape(_NT, _PF2)

    return {
        "idx1": i32(feat_rows(f1_pix).reshape(_NT, _PF1 * _C)),
        "idx2": i32(feat_rows(f2_pix).reshape(_NT, _PF2 * _C)),
        "lane1": i32(np.broadcast_to(
            feat_lanes(f1_pix).reshape(_NT, _PF1, 1, 1), (_NT, _PF1, 3, 16)
        ).reshape(_NT, _PF1 * 48)),
        "lane2": i32(np.broadcast_to(
            feat_lanes(f2_pix).reshape(_NT, _PF2, 1, 1), (_NT, _PF2, 3, 16)
        ).reshape(_NT, _PF2 * 48)),
        "labmeta": i32(labmeta),
    }


def _sc_gather_body(
    f1_tab, f2_tab, l1_tab, l2_tab,
    idx1_h, idx2_h, lane1_h, lane2_h, labmeta_h,
    g_out, lab_out,
    idx1_v, idx2_v, lane1_v, lane2_v, labmeta_v,
    rows0_v, rows1_v, rows2_v, labrows1_v, labrows2_v,
    out1_v, out2_v, labout_v, s0, s1, s2, sl1, sl2,
):
    wid = lax.axis_index("s") * 2 + lax.axis_index("c")
    iota = lax.iota(jnp.int32, 16)
    rings = (rows0_v, rows1_v, rows2_v)
    sems = (s0, s1, s2)

    pltpu.sync_copy(idx1_h.at[wid], idx1_v)
    pltpu.sync_copy(idx2_h.at[wid], idx2_v)
    pltpu.sync_copy(lane1_h.at[wid], lane1_v)
    pltpu.sync_copy(lane2_h.at[wid], lane2_v)
    pltpu.sync_copy(labmeta_h.at[wid], labmeta_v)

    dl1 = pltpu.async_copy(l1_tab.at[labmeta_v.at[pl.ds(0, _PF1)]],
                           labrows1_v, sl1)
    dl2 = pltpu.async_copy(l2_tab.at[labmeta_v.at[pl.ds(_PF1, _PF2)]],
                           labrows2_v, sl2)

    def issue(tab, idx_v, c, b):
        pltpu.async_copy(
            tab.at[idx_v.at[pl.ds(c * 128, 128)]], rings[b], sems[b])

    def drain(tab, idx_v, c, b):
        pltpu.make_async_copy(
            tab.at[idx_v.at[pl.ds(c * 128, 128)]], rings[b], sems[b]).wait()

    def stream_table(tab, idx_v, lane_v, out_v, n_chunks):
        issue(tab, idx_v, 0, 0)
        issue(tab, idx_v, 1, 1)

        def tri(i, carry):
            c0 = i * 3
            for b in range(3):
                c = c0 + b

                @pl.when(c + 2 < n_chunks)
                def _():
                    issue(tab, idx_v, c + 2, (b + 2) % 3)

                drain(tab, idx_v, c, b)
                lanes = lane_v[pl.ds(c * 16, 16)]
                ring_b = rings[b]

                def ext(g, carry2, ring_b=ring_b, c=c, lanes=lanes):
                    vals = plsc.load_gather(ring_b, [g * 16 + iota, lanes])
                    out_v[pl.ds(c * 128 + g * 16, 16)] = vals
                    return carry2

                lax.fori_loop(0, 8, ext, 0)
            return carry

        lax.fori_loop(0, n_chunks // 3, tri, 0)

    stream_table(f1_tab, idx1_v, lane1_v, out1_v, 3 * _PF1)
    stream_table(f2_tab, idx2_v, lane2_v, out2_v, 3 * _PF2)

    pltpu.sync_copy(out1_v, g_out.at[pl.ds(wid * (_PF1 * _C), _PF1 * _C)])
    pltpu.sync_copy(
        out2_v, g_out.at[pl.ds(1024 * _C + wid * (_PF2 * _C), _PF2 * _C)])

    dl1.wait()
    dl2.wait()
    for g in range(_PF1 // 16):
        lanes = labmeta_v[pl.ds(48 + g * 16, 16)]
        labout_v[pl.ds(g * 16, 16)] = plsc.load_gather(
            labrows1_v, [g * 16 + iota, lanes])
    for g in range(_PF2 // 16):
        lanes = labmeta_v[pl.ds(80 + g * 16, 16)]
        labout_v[pl.ds(_PF1 + g * 16, 16)] = plsc.load_gather(
            labrows2_v, [g * 16 + iota, lanes])
    pltpu.sync_copy(labout_v, lab_out.at[wid])


@functools.lru_cache(maxsize=1)
def _build_sc_gather():
    mesh = plsc.VectorSubcoreMesh(core_axis_name="c", subcore_axis_name="s")
    return pl.kernel(
        _sc_gather_body,
        out_type=(
            jax.ShapeDtypeStruct((1536 * _C,), jnp.float32),
            jax.ShapeDtypeStruct((_NT, 128), jnp.int32),
        ),
        mesh=mesh,
        compiler_params=pltpu.CompilerParams(needs_layout_passes=False),
        scratch_types=[
            pltpu.VMEM((_PF1 * _C,), jnp.int32),
            pltpu.VMEM((_PF2 * _C,), jnp.int32),
            pltpu.VMEM((_PF1 * 48,), jnp.int32),
            pltpu.VMEM((_PF2 * 48,), jnp.int32),
            pltpu.VMEM((128,), jnp.int32),
            pltpu.VMEM((128, _W), jnp.float32),
            pltpu.VMEM((128, _W), jnp.float32),
            pltpu.VMEM((128, _W), jnp.float32),
            pltpu.VMEM((_PF1, 512), jnp.int32),
            pltpu.VMEM((_PF2, 512), jnp.int32),
            pltpu.VMEM((_PF1 * _C,), jnp.float32),
            pltpu.VMEM((_PF2 * _C,), jnp.float32),
            pltpu.VMEM((128,), jnp.int32),
            pltpu.SemaphoreType.DMA,
            pltpu.SemaphoreType.DMA,
            pltpu.SemaphoreType.DMA,
            pltpu.SemaphoreType.DMA,
            pltpu.SemaphoreType.DMA,
        ],
    )


def _tc_loss_body(g_ref, alab_ref, clab_ref, out_ref):
    g = g_ref[...]
    ss = jnp.sum(g * g, axis=1, keepdims=True)
    norm = jnp.maximum(jnp.sqrt(ss), 1e-12)
    gn = g / norm
    anchors = gn[0:512, :]
    contras = gn[512:1536, :]
    logits = lax.dot_general(
        anchors, contras, (((1,), (1,)), ((), ())),
        preferred_element_type=jnp.float32,
    ) * (1.0 / _TEMPERATURE)

    a = alab_ref[...]
    c = clab_ref[...]
    eq = a == c
    ri = lax.broadcasted_iota(jnp.int32, (512, 1024), 0)
    ci = lax.broadcasted_iota(jnp.int32, (512, 1024), 1)
    mask = jnp.where(eq & (ri != ci), 1.0, 0.0)

    lmax = jnp.max(logits, axis=1, keepdims=True)
    l = logits - lmax
    e = jnp.exp(l)
    negsum = jnp.sum(e * (1.0 - mask), axis=1, keepdims=True)
    lp = l - jnp.log(e + negsum)
    msum = jnp.sum(mask, axis=1)
    msum = jnp.where(msum == 0.0, 1.0, msum)
    mlpp = jnp.sum(mask * lp, axis=1) / msum
    out_ref[0, 0] = -jnp.mean(mlpp)


def _tc_loss(g, a_lab, c_lab):
    return pl.pallas_call(
        _tc_loss_body,
        out_shape=jax.ShapeDtypeStruct((1, 1), jnp.float32),
        out_specs=pl.BlockSpec(memory_space=pltpu.SMEM),
    )(g, a_lab, c_lab)


def kernel(feat1, label1, feat2, label2):
    plan = _plan()
    g_flat, lab2d = _build_sc_gather()(
        feat1.reshape(-1, _W),
        feat2.reshape(-1, _W),
        label1.reshape(-1, 512),
        label2.reshape(-1, 512),
        plan["idx1"], plan["idx2"], plan["lane1"], plan["lane2"],
        plan["labmeta"],
    )
    g = g_flat.reshape(1536, _C)
    lab1 = lab2d[:, :_PF1].reshape(1024)
    lab2 = lab2d[:, _PF1:_PF1 + _PF2].reshape(512)
    a_lab = lab1[:512].astype(jnp.float32).reshape(512, 1)
    c_lab = jnp.concatenate([lab1[512:], lab2]).astype(
        jnp.float32).reshape(1, 1024)
    return _tc_loss(g, a_lab, c_lab)[0, 0]
